# baseline (speedup 1.0000x reference)
import functools

import jax
import jax.numpy as jnp
from jax import lax
from jax.experimental import pallas as pl
from jax.experimental.pallas import tpu as pltpu
from jax.experimental.pallas import tpu_sc as plsc

SUB = 16
NCORE = 2
CHUNK = 128
KSUB_A = 8
KSUB_B = 2
SUPER_A = CHUNK * KSUB_A
SUPER_B = CHUNK * KSUB_B


def _mm_norm(x, attn, weight, block_rows):
  n, k = x.shape
  out = weight.shape[1]
  grid = n // block_rows

  def body(x_ref, attn_ref, w_ref, h_ref, nrm_ref, ssq_ref):
    i = pl.program_id(0)

    @pl.when(i == 0)
    def _():
      ssq_ref[0] = 0.0

    h_ref[...] = lax.dot_general(
        x_ref[...], w_ref[...], (((1,), (0,)), ((), ())),
        precision=lax.Precision.DEFAULT,
        preferred_element_type=jnp.float32)
    a = attn_ref[...]
    ssq_ref[0] += jnp.sum(a * a)

    @pl.when(i == grid - 1)
    def _():
      nrm_ref[0, 0] = jnp.sqrt(ssq_ref[0])

  return pl.pallas_call(
      body,
      grid=(grid,),
      in_specs=[
          pl.BlockSpec((block_rows, k), lambda i: (i, 0)),
          pl.BlockSpec((block_rows, k), lambda i: (i, 0)),
          pl.BlockSpec((k, out), lambda i: (0, 0)),
      ],
      out_specs=[
          pl.BlockSpec((block_rows, out), lambda i: (i, 0)),
          pl.BlockSpec(memory_space=pltpu.SMEM),
      ],
      out_shape=[
          jax.ShapeDtypeStruct((n, out), jnp.float32),
          jax.ShapeDtypeStruct((1, 1), jnp.float32),
      ],
      scratch_shapes=[pltpu.SMEM((1,), jnp.float32)],
  )(x, attn, weight)


def _edge_vals_sc(src_r, dst_r, ev_r, attn_flat, n, nsuper):
  mesh = plsc.VectorSubcoreMesh(core_axis_name="c", subcore_axis_name="s")

  @functools.partial(
      pl.kernel,
      out_type=jax.ShapeDtypeStruct(
          (NCORE, SUB, nsuper, KSUB_A, CHUNK), jnp.float32),
      mesh=mesh,
      scratch_types=[
          pltpu.VMEM((KSUB_A, CHUNK), jnp.int32),
          pltpu.VMEM((KSUB_A, CHUNK), jnp.int32),
          pltpu.VMEM((KSUB_A, CHUNK), jnp.float32),
          pltpu.VMEM((KSUB_A, CHUNK), jnp.int32),
          pltpu.VMEM((KSUB_A, CHUNK), jnp.float32),
          pltpu.VMEM((KSUB_A, CHUNK), jnp.float32),
          pltpu.SemaphoreType.DMA,
          pltpu.SemaphoreType.DMA,
      ],
      compiler_params=pltpu.CompilerParams(use_tc_tiling_on_sc=False),
  )
  def k(src_hbm, dst_hbm, ev_hbm, attn_hbm, vals_hbm,
        src_v, dst_v, ev_v, fidx_v, dv_v, vals_v, sem, sem2):
    c = lax.axis_index("c")
    s = lax.axis_index("s")

    def superchunk(g, carry):
      d1 = pltpu.async_copy(src_hbm.at[c, s, g], src_v, sem)
      d2 = pltpu.async_copy(dst_hbm.at[c, s, g], dst_v, sem)
      d3 = pltpu.async_copy(ev_hbm.at[c, s, g], ev_v, sem)
      d1.wait(); d2.wait(); d3.wait()

      def fidx_body(t, _):
        j = t // (CHUNK // 16)
        o = (t % (CHUNK // 16)) * 16
        sv = src_v[j, pl.ds(o, 16)]
        dv = dst_v[j, pl.ds(o, 16)]
        fidx_v[j, pl.ds(o, 16)] = sv * n + dv
        return 0
      lax.fori_loop(0, KSUB_A * CHUNK // 16, fidx_body, 0)

      descs = []
      for j in range(KSUB_A):
        descs.append(pltpu.async_copy(
            attn_hbm.at[fidx_v.at[j]], dv_v.at[j], sem))
      for d in descs:
        d.wait()

      def vals_body(t, _):
        j = t // (CHUNK // 16)
        o = (t % (CHUNK // 16)) * 16
        vals_v[j, pl.ds(o, 16)] = (
            ev_v[j, pl.ds(o, 16)] * dv_v[j, pl.ds(o, 16)])
        return 0
      lax.fori_loop(0, KSUB_A * CHUNK // 16, vals_body, 0)

      pltpu.sync_copy(vals_v, vals_hbm.at[c, s, g])
      return carry

    lax.fori_loop(0, nsuper, superchunk, 0)

  return k(src_r, dst_r, ev_r, attn_flat)


def _edge_scatter_sc(src_r, dst_r, vals_r, h, zrows, n, out, nsuper):
  rpt = n // SUB
  mesh = plsc.VectorSubcoreMesh(core_axis_name="c", subcore_axis_name="s")

  @functools.partial(
      pl.kernel,
      out_type=jax.ShapeDtypeStruct((NCORE * SUB, n // SUB, out),
                                    jnp.float32),
      mesh=mesh,
      scratch_types=[
          pltpu.VMEM_SHARED((n, out), jnp.float32),
          pltpu.VMEM((KSUB_B, CHUNK), jnp.int32),
          pltpu.VMEM((KSUB_B, CHUNK), jnp.int32),
          pltpu.VMEM((KSUB_B, CHUNK), jnp.float32),
          pltpu.VMEM((SUPER_B, out), jnp.float32),
          pltpu.SemaphoreType.DMA,
          pltpu.SemaphoreType.DMA,
      ],
      compiler_params=pltpu.CompilerParams(use_tc_tiling_on_sc=False),
  )
  def k(src_hbm, dst_hbm, vals_hbm, h_hbm, z_hbm, out_hbm,
        acc, src_v, dst_v, vals_v, rows_v, sem, sem2):
    c = lax.axis_index("c")
    s = lax.axis_index("s")

    pltpu.sync_copy(z_hbm, acc.at[pl.ds(s * rpt, rpt), :])
    plsc.subcore_barrier()

    def superchunk(g, carry):
      d1 = pltpu.async_copy(src_hbm.at[c, s, g], src_v, sem)
      d2 = pltpu.async_copy(dst_hbm.at[c, s, g], dst_v, sem)
      d3 = pltpu.async_copy(vals_hbm.at[c, s, g], vals_v, sem)
      d1.wait(); d2.wait(); d3.wait()

      descs = []
      for j in range(KSUB_B):
        descs.append(pltpu.async_copy(
            h_hbm.at[dst_v.at[j]],
            rows_v.at[pl.ds(j * CHUNK, CHUNK), :], sem))
      for d in descs:
        d.wait()

      def scale_body(g2, _):
        val16 = vals_v[g2 // (CHUNK // 16),
                       pl.ds((g2 % (CHUNK // 16)) * 16, 16)]
        base = g2 * 16
        for lane in range(16):
          valv = jnp.broadcast_to(val16[lane], (16,))
          for f in range(out // 16):
            sl = pl.ds(f * 16, 16)
            rows_v[base + lane, sl] = rows_v[base + lane, sl] * valv
        return 0
      lax.fori_loop(0, SUPER_B // 16, scale_body, 0)

      sdescs = []
      for j in range(KSUB_B):
        sdescs.append(pltpu.async_copy(
            rows_v.at[pl.ds(j * CHUNK, CHUNK), :],
            acc.at[src_v.at[j]], sem2, add=True))
      for d in sdescs:
        d.wait()
      return carry

    lax.fori_loop(0, nsuper, superchunk, 0)
    plsc.subcore_barrier()
    pltpu.sync_copy(acc.at[pl.ds(s * rpt, rpt), :],
                    out_hbm.at[c * SUB + s])

  return k(src_r, dst_r, vals_r, h, zrows)


def _combine(p0, p1, block_rows):
  n, out = p0.shape
  grid = n // block_rows

  def body(a_ref, b_ref, o_ref):
    o_ref[...] = a_ref[...] + b_ref[...]

  return pl.pallas_call(
      body,
      grid=(grid,),
      in_specs=[
          pl.BlockSpec((block_rows, out), lambda i: (i, 0)),
          pl.BlockSpec((block_rows, out), lambda i: (i, 0)),
      ],
      out_specs=pl.BlockSpec((block_rows, out), lambda i: (i, 0)),
      out_shape=jax.ShapeDtypeStruct((n, out), jnp.float32),
  )(p0, p1)


def kernel(x, edge_index, edge_values, attn, weight):
  n = x.shape[0]
  out = weight.shape[1]
  e = edge_values.shape[0]

  unit = NCORE * SUB * SUPER_A
  e_pad = ((e + unit - 1) // unit) * unit
  pad = e_pad - e
  ns_a = e_pad // (NCORE * SUB * SUPER_A)
  ns_b = e_pad // (NCORE * SUB * SUPER_B)
  src = jnp.concatenate([edge_index[0], jnp.zeros((pad,), jnp.int32)])
  dst = jnp.concatenate([edge_index[1], jnp.zeros((pad,), jnp.int32)])
  ev = jnp.concatenate([edge_values, jnp.zeros((pad,), jnp.float32)])
  src_a = src.reshape(NCORE, SUB, ns_a, KSUB_A, CHUNK)
  dst_a = dst.reshape(NCORE, SUB, ns_a, KSUB_A, CHUNK)
  ev_a = ev.reshape(NCORE, SUB, ns_a, KSUB_A, CHUNK)
  src_b = src.reshape(NCORE, SUB, ns_b, KSUB_B, CHUNK)
  dst_b = dst.reshape(NCORE, SUB, ns_b, KSUB_B, CHUNK)

  attn_flat = attn.reshape(-1)
  zrows = jnp.zeros((n // SUB, out), jnp.float32)

  vals = _edge_vals_sc(src_a, dst_a, ev_a, attn_flat, n, ns_a)
  h, nrm = _mm_norm(x.astype(jnp.float32), attn, weight, block_rows=200)
  vals_b = vals.reshape(NCORE, SUB, ns_b, KSUB_B, CHUNK)

  parts = _edge_scatter_sc(src_b, dst_b, vals_b, h, zrows, n, out, ns_b)
  parts = parts.reshape(NCORE, n, out)
  result = _combine(parts[0], parts[1], block_rows=1000)
  return result, nrm[0, 0]


# speedup vs baseline: 1.0170x; 1.0170x over previous
import functools

import jax
import jax.numpy as jnp
from jax import lax
from jax.experimental import pallas as pl
from jax.experimental.pallas import tpu as pltpu
from jax.experimental.pallas import tpu_sc as plsc

SUB = 16
NCORE = 2
CHUNK = 64
KSUB = 4
SUPER = CHUNK * KSUB


def _mm_norm(x, attn, weight, block_rows):
  n, k = x.shape
  out = weight.shape[1]
  grid = n // block_rows

  def body(x_ref, attn_ref, w_ref, h_ref, nrm_ref, ssq_ref):
    i = pl.program_id(0)

    @pl.when(i == 0)
    def _():
      ssq_ref[0] = 0.0

    h_ref[...] = lax.dot_general(
        x_ref[...], w_ref[...], (((1,), (0,)), ((), ())),
        precision=lax.Precision.DEFAULT,
        preferred_element_type=jnp.float32)
    a = attn_ref[...]
    ssq_ref[0] += jnp.sum(a * a)

    @pl.when(i == grid - 1)
    def _():
      nrm_ref[0, 0] = jnp.sqrt(ssq_ref[0])

  return pl.pallas_call(
      body,
      grid=(grid,),
      in_specs=[
          pl.BlockSpec((block_rows, k), lambda i: (i, 0)),
          pl.BlockSpec((block_rows, k), lambda i: (i, 0)),
          pl.BlockSpec((k, out), lambda i: (0, 0)),
      ],
      out_specs=[
          pl.BlockSpec((block_rows, out), lambda i: (i, 0)),
          pl.BlockSpec(memory_space=pltpu.SMEM),
      ],
      out_shape=[
          jax.ShapeDtypeStruct((n, out), jnp.float32),
          jax.ShapeDtypeStruct((1, 1), jnp.float32),
      ],
      scratch_shapes=[pltpu.SMEM((1,), jnp.float32)],
  )(x, attn, weight)


def _edge_sc(src_r, dst_r, ev_r, attn_flat, h, zrows, n, out, nsuper):
  rpt = n // SUB
  mesh = plsc.VectorSubcoreMesh(core_axis_name="c", subcore_axis_name="s")

  @functools.partial(
      pl.kernel,
      out_type=jax.ShapeDtypeStruct((NCORE * SUB, n // SUB, out),
                                    jnp.float32),
      mesh=mesh,
      scratch_types=[
          pltpu.VMEM_SHARED((n, out), jnp.float32),
          pltpu.VMEM((KSUB, CHUNK), jnp.int32),
          pltpu.VMEM((KSUB, CHUNK), jnp.int32),
          pltpu.VMEM((KSUB, CHUNK), jnp.float32),
          pltpu.VMEM((KSUB, CHUNK), jnp.int32),
          pltpu.VMEM((KSUB, CHUNK), jnp.float32),
          pltpu.VMEM((SUPER,), jnp.float32),
          pltpu.VMEM((SUPER, out), jnp.float32),
          pltpu.SemaphoreType.DMA,
          pltpu.SemaphoreType.DMA,
      ],
      compiler_params=pltpu.CompilerParams(use_tc_tiling_on_sc=False),
  )
  def k(src_hbm, dst_hbm, ev_hbm, attn_hbm, h_hbm, z_hbm, out_hbm,
        acc, src_v, dst_v, ev_v, fidx_v, dv_v, vals_v, rows_v,
        sem, sem2):
    c = lax.axis_index("c")
    s = lax.axis_index("s")

    pltpu.sync_copy(z_hbm, acc.at[pl.ds(s * rpt, rpt), :])
    plsc.subcore_barrier()

    def superchunk(g, carry):
      d1 = pltpu.async_copy(src_hbm.at[c, s, g], src_v, sem)
      d2 = pltpu.async_copy(dst_hbm.at[c, s, g], dst_v, sem)
      d3 = pltpu.async_copy(ev_hbm.at[c, s, g], ev_v, sem)
      d1.wait(); d2.wait(); d3.wait()

      def fidx_body(t, _):
        j = t // (CHUNK // 16)
        o = (t % (CHUNK // 16)) * 16
        sv = src_v[j, pl.ds(o, 16)]
        dv = dst_v[j, pl.ds(o, 16)]
        fidx_v[j, pl.ds(o, 16)] = sv * n + dv
        return 0
      lax.fori_loop(0, KSUB * CHUNK // 16, fidx_body, 0)

      descs = []
      for j in range(KSUB):
        descs.append(pltpu.async_copy(
            attn_hbm.at[fidx_v.at[j]], dv_v.at[j], sem))
      for j in range(KSUB):
        descs.append(pltpu.async_copy(
            h_hbm.at[dst_v.at[j]],
            rows_v.at[pl.ds(j * CHUNK, CHUNK), :], sem))
      for d in descs:
        d.wait()

      def vals_body(t, _):
        j = t // (CHUNK // 16)
        o = (t % (CHUNK // 16)) * 16
        vals_v[pl.ds(t * 16, 16)] = (
            ev_v[j, pl.ds(o, 16)] * dv_v[j, pl.ds(o, 16)])
        return 0
      lax.fori_loop(0, KSUB * CHUNK // 16, vals_body, 0)

      def scale_body(g2, _):
        val16 = vals_v[pl.ds(g2 * 16, 16)]
        base = g2 * 16
        for lane in range(16):
          valv = jnp.broadcast_to(val16[lane], (16,))
          for f in range(out // 16):
            sl = pl.ds(f * 16, 16)
            rows_v[base + lane, sl] = rows_v[base + lane, sl] * valv
        return 0
      lax.fori_loop(0, SUPER // 16, scale_body, 0)

      sdescs = []
      for j in range(KSUB):
        sdescs.append(pltpu.async_copy(
            rows_v.at[pl.ds(j * CHUNK, CHUNK), :],
            acc.at[src_v.at[j]], sem2, add=True))
      for d in sdescs:
        d.wait()
      return carry

    lax.fori_loop(0, nsuper, superchunk, 0)
    plsc.subcore_barrier()
    pltpu.sync_copy(acc.at[pl.ds(s * rpt, rpt), :],
                    out_hbm.at[c * SUB + s])

  return k(src_r, dst_r, ev_r, attn_flat, h, zrows)


def _combine(p0, p1, block_rows):
  n, out = p0.shape
  grid = n // block_rows

  def body(a_ref, b_ref, o_ref):
    o_ref[...] = a_ref[...] + b_ref[...]

  return pl.pallas_call(
      body,
      grid=(grid,),
      in_specs=[
          pl.BlockSpec((block_rows, out), lambda i: (i, 0)),
          pl.BlockSpec((block_rows, out), lambda i: (i, 0)),
      ],
      out_specs=pl.BlockSpec((block_rows, out), lambda i: (i, 0)),
      out_shape=jax.ShapeDtypeStruct((n, out), jnp.float32),
  )(p0, p1)


def kernel(x, edge_index, edge_values, attn, weight):
  n = x.shape[0]
  out = weight.shape[1]
  e = edge_values.shape[0]

  h, nrm = _mm_norm(x.astype(jnp.float32), attn, weight, block_rows=200)

  unit = NCORE * SUB * SUPER
  e_pad = ((e + unit - 1) // unit) * unit
  nsuper = e_pad // unit
  pad = e_pad - e
  src = jnp.concatenate([edge_index[0], jnp.zeros((pad,), jnp.int32)])
  dst = jnp.concatenate([edge_index[1], jnp.zeros((pad,), jnp.int32)])
  ev = jnp.concatenate([edge_values, jnp.zeros((pad,), jnp.float32)])
  src_r = src.reshape(NCORE, SUB, nsuper, KSUB, CHUNK)
  dst_r = dst.reshape(NCORE, SUB, nsuper, KSUB, CHUNK)
  ev_r = ev.reshape(NCORE, SUB, nsuper, KSUB, CHUNK)

  attn_flat = attn.reshape(-1)
  zrows = jnp.zeros((n // SUB, out), jnp.float32)

  parts = _edge_sc(src_r, dst_r, ev_r, attn_flat, h, zrows,
                   n, out, nsuper)
  parts = parts.reshape(NCORE, n, out)
  result = _combine(parts[0], parts[1], block_rows=1000)
  return result, nrm[0, 0]
